# baseline (speedup 1.0000x reference)
import functools

import jax
import jax.numpy as jnp
from jax import lax
from jax.experimental import pallas as pl

_STEEPNESS = 12.0
_NB = 8


def _fused_body(x1_ref, w_ref, x0_ref, out_ref, *, D):
    nb, C, Dr = x1_ref.shape
    F = x0_ref.shape[-1]

    cr = jnp.sum(x1_ref[...], axis=-1) * w_ref[0][None, :]

    a = cr[:, :, None]
    b = cr[:, None, :]
    j_idx = lax.broadcasted_iota(jnp.int32, (1, C, C), 2)
    i_idx = lax.broadcasted_iota(jnp.int32, (1, C, C), 1)
    before = (b > a) | ((b == a) & (j_idx < i_idx))
    rank = jnp.sum(before.astype(jnp.float32), axis=2)
    mask = jnp.exp((-_STEEPNESS / (C - 1)) * rank)

    col = lax.broadcasted_iota(jnp.int32, (1, 1, F), 2)
    is_masked = (col % D) == (D - 1)
    x0 = x0_ref[...]
    out_ref[...] = jnp.where(is_masked, x0 * mask[:, :, None], x0)


def kernel(x0, x1, boosting_weights):
    B, C, H, W, D = x0.shape
    F = H * W * D
    x0r = x0.reshape(B, C, F)
    w = boosting_weights.reshape(1, C)
    out = pl.pallas_call(
        functools.partial(_fused_body, D=D),
        grid=(B // _NB,),
        in_specs=[
            pl.BlockSpec((_NB, C, x1.shape[-1]), lambda i: (i, 0, 0)),
            pl.BlockSpec((1, C), lambda i: (0, 0)),
            pl.BlockSpec((_NB, C, F), lambda i: (i, 0, 0)),
        ],
        out_specs=pl.BlockSpec((_NB, C, F), lambda i: (i, 0, 0)),
        out_shape=jax.ShapeDtypeStruct((B, C, F), x0.dtype),
    )(x1, w, x0r)
    return out.reshape(B, C, H, W, D)


# speedup vs baseline: 2.1084x; 2.1084x over previous
import functools

import jax
import jax.numpy as jnp
from jax import lax
from jax.experimental import pallas as pl
from jax.experimental.pallas import tpu as pltpu

_STEEPNESS = 12.0
_NB = 8


def _fused_body(x1_ref, w_ref, x0_ref, out_ref, *, D):
    nb, C, Dr = x1_ref.shape
    F = x0_ref.shape[-1]

    cr = jnp.sum(x1_ref[...], axis=-1) * w_ref[...]

    lane = lax.broadcasted_iota(jnp.int32, (nb, C), 1)
    rank = jnp.zeros((nb, C), jnp.float32)
    for r in range(1, C):
        crj = pltpu.roll(cr, C - r, axis=1)
        gt = crj > cr
        tie = (crj == cr) & (lane >= C - r)
        rank = rank + (gt | tie).astype(jnp.float32)
    mask = jnp.exp((-_STEEPNESS / (C - 1)) * rank)
    mask_t = mask.T

    col = lax.broadcasted_iota(jnp.int32, (C, F), 1)
    is_masked = (col % D) == (D - 1)
    for b in range(nb):
        mb = lax.broadcast_in_dim(mask_t[:, b:b + 1], (C, F), (0, 1))
        xb = x0_ref[b]
        out_ref[b] = jnp.where(is_masked, xb * mb, xb)


def kernel(x0, x1, boosting_weights):
    B, C, H, W, D = x0.shape
    F = H * W * D
    x0r = x0.reshape(B, C, F)
    w = boosting_weights.reshape(1, C)
    out = pl.pallas_call(
        functools.partial(_fused_body, D=D),
        grid=(B // _NB,),
        in_specs=[
            pl.BlockSpec((_NB, C, x1.shape[-1]), lambda i: (i, 0, 0)),
            pl.BlockSpec((1, C), lambda i: (0, 0)),
            pl.BlockSpec((_NB, C, F), lambda i: (i, 0, 0)),
        ],
        out_specs=pl.BlockSpec((_NB, C, F), lambda i: (i, 0, 0)),
        out_shape=jax.ShapeDtypeStruct((B, C, F), x0.dtype),
    )(x1, w, x0r)
    return out.reshape(B, C, H, W, D)


# speedup vs baseline: 13.4288x; 6.3691x over previous
import functools

import jax
import jax.numpy as jnp
from jax import lax
from jax.experimental import pallas as pl
from jax.experimental.pallas import tpu as pltpu

_STEEPNESS = 12.0
_NB = 8


def _fused_body(x1_ref, w_ref, x0_ref, out_ref):
    nb, R, C = x1_ref.shape
    D = x0_ref.shape[2]

    cr = jnp.sum(x1_ref[...], axis=1) * w_ref[...]

    lane = lax.broadcasted_iota(jnp.int32, (nb, C), 1)
    rank = jnp.zeros((nb, C), jnp.float32)
    for r in range(1, C):
        crj = pltpu.roll(cr, C - r, axis=1)
        gt = crj > cr
        tie = (crj == cr) & (lane >= C - r)
        rank = rank + (gt | tie).astype(jnp.float32)
    mask = jnp.exp((-_STEEPNESS / (C - 1)) * rank)

    out_ref[:, :, 0:D - 1] = x0_ref[:, :, 0:D - 1]
    out_ref[:, :, D - 1:D] = (
        x0_ref[:, :, D - 1:D] * mask[:, None, None, None, :]
    )


def kernel(x0, x1, boosting_weights):
    B, C, H, W, D = x0.shape
    xt = lax.transpose(x0, (0, 2, 4, 3, 1))
    x1t = lax.transpose(x1, (0, 2, 1))
    R = x1t.shape[1]
    w = boosting_weights.reshape(1, C)
    out = pl.pallas_call(
        _fused_body,
        grid=(B // _NB,),
        in_specs=[
            pl.BlockSpec((_NB, R, C), lambda i: (i, 0, 0)),
            pl.BlockSpec((1, C), lambda i: (0, 0)),
            pl.BlockSpec((_NB, H, D, W, C), lambda i: (i, 0, 0, 0, 0)),
        ],
        out_specs=pl.BlockSpec((_NB, H, D, W, C), lambda i: (i, 0, 0, 0, 0)),
        out_shape=jax.ShapeDtypeStruct((B, H, D, W, C), x0.dtype),
    )(x1t, w, xt)
    return lax.transpose(out, (0, 4, 1, 3, 2))


# speedup vs baseline: 13.4488x; 1.0015x over previous
import functools

import jax
import jax.numpy as jnp
from jax import lax
from jax.experimental import pallas as pl
from jax.experimental.pallas import tpu as pltpu

_STEEPNESS = 12.0
_NB = 16


def _fused_body(x1_ref, w_ref, x0_ref, out_ref):
    nb, R, C = x1_ref.shape
    D = x0_ref.shape[2]

    cr = jnp.sum(x1_ref[...], axis=1) * w_ref[...]

    lane = lax.broadcasted_iota(jnp.int32, (nb, C), 1)
    rank = jnp.zeros((nb, C), jnp.float32)
    for r in range(1, C):
        crj = pltpu.roll(cr, C - r, axis=1)
        gt = crj > cr
        tie = (crj == cr) & (lane >= C - r)
        rank = rank + (gt | tie).astype(jnp.float32)
    mask = jnp.exp((-_STEEPNESS / (C - 1)) * rank)

    out_ref[:, :, 0:D - 1] = x0_ref[:, :, 0:D - 1]
    out_ref[:, :, D - 1:D] = (
        x0_ref[:, :, D - 1:D] * mask[:, None, None, None, :]
    )


def kernel(x0, x1, boosting_weights):
    B, C, H, W, D = x0.shape
    xt = lax.transpose(x0, (0, 2, 4, 3, 1))
    x1t = lax.transpose(x1, (0, 2, 1))
    R = x1t.shape[1]
    w = boosting_weights.reshape(1, C)
    out = pl.pallas_call(
        _fused_body,
        grid=(B // _NB,),
        in_specs=[
            pl.BlockSpec((_NB, R, C), lambda i: (i, 0, 0)),
            pl.BlockSpec((1, C), lambda i: (0, 0)),
            pl.BlockSpec((_NB, H, D, W, C), lambda i: (i, 0, 0, 0, 0)),
        ],
        out_specs=pl.BlockSpec((_NB, H, D, W, C), lambda i: (i, 0, 0, 0, 0)),
        out_shape=jax.ShapeDtypeStruct((B, H, D, W, C), x0.dtype),
    )(x1t, w, xt)
    return lax.transpose(out, (0, 4, 1, 3, 2))
